# baseline (speedup 1.0000x reference)
import functools

import jax
import jax.numpy as jnp
from jax import lax
from jax.experimental import pallas as pl
from jax.experimental.pallas import tpu as pltpu
from jax.experimental.pallas import tpu_sc as plsc

H = 128
N_MOL = 1024
APM = 48
N_ATOMS = 1 + N_MOL * APM
N_BONDS = 1 + N_MOL * APM * 4
MAX_NB = 6
REAL_A = N_MOL * APM
ORIG0 = REAL_A

NC, NS = 2, 16
NW = NC * NS

CA = 128
PAD_A = 49280
NCHT_A = PAD_A // CA

CB = 128
PAD_B = 196736
NCHT_B = PAD_B // CB

_MESH = plsc.VectorSubcoreMesh(core_axis_name="c", subcore_axis_name="s")


def _wid():
    return lax.axis_index("s") * NC + lax.axis_index("c")


def _sc_atom_agg_body(mb, a2bt, ma, ridx, out, idx_v, ridx_v, rows_v, ma_v,
                      sem):
    w = _wid()
    nch = (NCHT_A - w + NW - 1) // NW

    def chunk(ci, _):
        ao = (w + ci * NW) * CA
        pltpu.sync_copy(a2bt.at[:, pl.ds(ao, CA)], idx_v)
        pltpu.sync_copy(ridx.at[pl.ds(ao, CA)], ridx_v)
        cps = [pltpu.async_copy(mb.at[idx_v.at[j]], rows_v.at[j], sem)
               for j in range(MAX_NB)]
        cps.append(pltpu.async_copy(ma.at[ridx_v], ma_v, sem))
        for cp in cps:
            cp.wait()

        def row(r, _):
            for k in range(H // 16):
                sl = pl.ds(k * 16, 16)
                v = rows_v[0, r, sl]
                s_, m_ = v, v
                for j in range(1, MAX_NB):
                    vj = rows_v[j, r, sl]
                    s_ = s_ + vj
                    m_ = jnp.maximum(m_, vj)
                ma_v[r, sl] = ma_v[r, sl] + s_ * m_
            return 0

        lax.fori_loop(0, CA, row, 0)
        pltpu.sync_copy(ma_v, out.at[pl.ds(ao, CA)])
        return 0

    lax.fori_loop(0, nch, chunk, 0)


_sc_atom_agg = functools.partial(
    pl.kernel,
    out_type=jax.ShapeDtypeStruct((PAD_A, H), jnp.float32),
    mesh=_MESH,
    scratch_types=[
        pltpu.VMEM((MAX_NB, CA), jnp.int32),
        pltpu.VMEM((CA,), jnp.int32),
        pltpu.VMEM((MAX_NB, CA, H), jnp.float32),
        pltpu.VMEM((CA, H), jnp.float32),
        pltpu.SemaphoreType.DMA,
    ],
)(_sc_atom_agg_body)


def _sc_atom_agg_raw_body(mb, a2bt, ia, ridx, out, ia_roll, idx_v, ridx_v,
                          rows_v, ia_v, sem):
    w = _wid()
    nch = (NCHT_A - w + NW - 1) // NW

    def chunk(ci, _):
        ao = (w + ci * NW) * CA
        pltpu.sync_copy(a2bt.at[:, pl.ds(ao, CA)], idx_v)
        pltpu.sync_copy(ridx.at[pl.ds(ao, CA)], ridx_v)
        cps = [pltpu.async_copy(mb.at[idx_v.at[j]], rows_v.at[j], sem)
               for j in range(MAX_NB)]
        cps.append(pltpu.async_copy(ia.at[ridx_v], ia_v, sem))
        for cp in cps:
            cp.wait()

        def row(r, _):
            for k in range(H // 16):
                sl = pl.ds(k * 16, 16)
                v = rows_v[0, r, sl]
                s_, m_ = v, v
                for j in range(1, MAX_NB):
                    vj = rows_v[j, r, sl]
                    s_ = s_ + vj
                    m_ = jnp.maximum(m_, vj)
                rows_v[0, r, sl] = s_ * m_
            return 0

        lax.fori_loop(0, CA, row, 0)
        pltpu.sync_copy(rows_v.at[0], out.at[pl.ds(ao, CA)])
        pltpu.sync_copy(ia_v, ia_roll.at[pl.ds(ao, CA)])
        return 0

    lax.fori_loop(0, nch, chunk, 0)


_sc_atom_agg_raw = functools.partial(
    pl.kernel,
    out_type=[jax.ShapeDtypeStruct((PAD_A, H), jnp.float32),
              jax.ShapeDtypeStruct((PAD_A, H), jnp.float32)],
    mesh=_MESH,
    scratch_types=[
        pltpu.VMEM((MAX_NB, CA), jnp.int32),
        pltpu.VMEM((CA,), jnp.int32),
        pltpu.VMEM((MAX_NB, CA, H), jnp.float32),
        pltpu.VMEM((CA, H), jnp.float32),
        pltpu.SemaphoreType.DMA,
    ],
)(_sc_atom_agg_raw_body)


def _sc_bond_msg_body(ma, mb, b2a, b2revb, out, ia_v, ir_v, ra_v, rb_v,
                      out_v, sem):
    w = _wid()
    nch = (NCHT_B - w + NW - 1) // NW

    def chunk(ci, _):
        bo = (w + ci * NW) * CB
        pltpu.sync_copy(b2a.at[pl.ds(bo, CB)], ia_v)
        pltpu.sync_copy(b2revb.at[pl.ds(bo, CB)], ir_v)
        c1 = pltpu.async_copy(ma.at[ia_v], ra_v, sem)
        c2 = pltpu.async_copy(mb.at[ir_v], rb_v, sem)
        c1.wait()
        c2.wait()

        def row(r, _):
            for k in range(H // 16):
                sl = pl.ds(k * 16, 16)
                out_v[r, sl] = ra_v[r, sl] - rb_v[r, sl]
            return 0

        lax.fori_loop(0, CB, row, 0)
        pltpu.sync_copy(out_v, out.at[pl.ds(bo, CB)])
        return 0

    lax.fori_loop(0, nch, chunk, 0)


_sc_bond_msg = functools.partial(
    pl.kernel,
    out_type=jax.ShapeDtypeStruct((PAD_B, H), jnp.float32),
    mesh=_MESH,
    scratch_types=[
        pltpu.VMEM((CB,), jnp.int32),
        pltpu.VMEM((CB,), jnp.int32),
        pltpu.VMEM((CB, H), jnp.float32),
        pltpu.VMEM((CB, H), jnp.float32),
        pltpu.VMEM((CB, H), jnp.float32),
        pltpu.SemaphoreType.DMA,
    ],
)(_sc_bond_msg_body)


def _dot_t(x, w):
    return lax.dot_general(x, w, (((1,), (1,)), ((), ())),
                           precision=None,
                           preferred_element_type=jnp.float32)


def _relu_mm_kernel(x_ref, w_ref, o_ref):
    o_ref[...] = jnp.maximum(_dot_t(x_ref[...], w_ref[...]), 0.0)


def _input_proj(x, w, br, n_out=None):
    n, k = x.shape
    if n_out is None:
        n_out = n
    return pl.pallas_call(
        _relu_mm_kernel,
        grid=(pl.cdiv(n_out, br),),
        in_specs=[pl.BlockSpec((br, k), lambda i: (i, 0)),
                  pl.BlockSpec((H, k), lambda i: (0, 0))],
        out_specs=pl.BlockSpec((br, H), lambda i: (i, 0)),
        out_shape=jax.ShapeDtypeStruct((n_out, H), jnp.float32),
    )(x, w)


def _bond_update_kernel(pre_ref, ib_ref, w_ref, o_ref):
    o_ref[...] = jnp.maximum(
        ib_ref[...] + _dot_t(pre_ref[...], w_ref[...]), 0.0)


def _bond_update(pre, ib, w_h):
    br = 2048
    return pl.pallas_call(
        _bond_update_kernel,
        grid=(pl.cdiv(PAD_B, br),),
        in_specs=[pl.BlockSpec((br, H), lambda i: (i, 0)),
                  pl.BlockSpec((br, H), lambda i: (i, 0)),
                  pl.BlockSpec((H, H), lambda i: (0, 0))],
        out_specs=pl.BlockSpec((br, H), lambda i: (i, 0)),
        out_shape=jax.ShapeDtypeStruct((PAD_B, H), jnp.float32),
    )(pre, ib, w_h)


def _node_kernel(agg_ref, ma_ref, ia_ref, w_ref, bias_ref, msg_ref, h0_ref):
    xcat = jnp.concatenate([agg_ref[...], ma_ref[...], ia_ref[...]], axis=1)
    node = _dot_t(xcat, w_ref[...])
    msg = jnp.maximum(node + bias_ref[...], 0.0)
    msg_ref[...] = msg.reshape(-1, APM, H).swapaxes(0, 1)
    h0_ref[...] = jnp.max(node.reshape(-1, APM, H), axis=1)


def _node_stage(agg, ma, ia, lr_w, bias):
    br = 768
    mols = br // APM
    return pl.pallas_call(
        _node_kernel,
        grid=(REAL_A // br,),
        in_specs=[pl.BlockSpec((br, H), lambda i: (i, 0)),
                  pl.BlockSpec((br, H), lambda i: (i, 0)),
                  pl.BlockSpec((br, H), lambda i: (i, 0)),
                  pl.BlockSpec((H, 3 * H), lambda i: (0, 0)),
                  pl.BlockSpec((1, H), lambda i: (0, 0))],
        out_specs=[pl.BlockSpec((APM, mols, H), lambda i: (0, i, 0)),
                   pl.BlockSpec((mols, H), lambda i: (i, 0))],
        out_shape=[jax.ShapeDtypeStruct((APM, N_MOL, H), jnp.float32),
                   jax.ShapeDtypeStruct((N_MOL, H), jnp.float32)],
    )(agg, ma, ia, lr_w, bias)


def _gru_step(x, h, wih, whh, bih, bhh):
    gi = _dot_t(x, wih) + bih
    gh = _dot_t(h, whh) + bhh
    r = jax.nn.sigmoid(gi[:, :H] + gh[:, :H])
    z = jax.nn.sigmoid(gi[:, H:2 * H] + gh[:, H:2 * H])
    n = jnp.tanh(gi[:, 2 * H:] + r * gh[:, 2 * H:])
    return (1.0 - z) * n + z * h


def _gru_kernel(xf_ref, xr_ref, h0_ref, wih_f, whh_f, bih_f, bhh_f,
                wih_r, whh_r, bih_r, bhh_r, of_ref, or_ref, hf, hr):
    t = pl.program_id(0)

    @pl.when(t == 0)
    def _():
        hf[...] = h0_ref[...]
        hr[...] = h0_ref[...]

    xf = xf_ref[...][0]
    hn_f = _gru_step(xf, hf[...], wih_f[...], whh_f[...], bih_f[...],
                     bhh_f[...])
    hf[...] = hn_f
    of_ref[...] = hn_f[None]

    xr = xr_ref[...][0]
    hn_r = _gru_step(xr, hr[...], wih_r[...], whh_r[...], bih_r[...],
                     bhh_r[...])
    hr[...] = hn_r
    or_ref[...] = hn_r[None]


def _gru(seq, h0, wih_f, whh_f, bih_f, bhh_f, wih_r, whh_r, bih_r, bhh_r):
    wspec = pl.BlockSpec((3 * H, H), lambda t: (0, 0))
    bspec = pl.BlockSpec((1, 3 * H), lambda t: (0, 0))
    return pl.pallas_call(
        _gru_kernel,
        grid=(APM,),
        in_specs=[pl.BlockSpec((1, N_MOL, H), lambda t: (t, 0, 0)),
                  pl.BlockSpec((1, N_MOL, H), lambda t: (APM - 1 - t, 0, 0)),
                  pl.BlockSpec((N_MOL, H), lambda t: (0, 0)),
                  wspec, wspec, bspec, bspec, wspec, wspec, bspec, bspec],
        out_specs=[pl.BlockSpec((1, N_MOL, H), lambda t: (t, 0, 0)),
                   pl.BlockSpec((1, N_MOL, H),
                                lambda t: (APM - 1 - t, 0, 0))],
        out_shape=[jax.ShapeDtypeStruct((APM, N_MOL, H), jnp.float32),
                   jax.ShapeDtypeStruct((APM, N_MOL, H), jnp.float32)],
        scratch_shapes=[pltpu.VMEM((N_MOL, H), jnp.float32),
                        pltpu.VMEM((N_MOL, H), jnp.float32)],
        compiler_params=pltpu.CompilerParams(
            dimension_semantics=("arbitrary",)),
    )(seq, seq, h0, wih_f, whh_f, bih_f, bhh_f, wih_r, whh_r, bih_r, bhh_r)


def _readout_kernel(of_ref, or_ref, w1_ref, w2_ref, b_ref, o_ref):
    bm = of_ref.shape[1]
    x = of_ref[...].reshape(APM * bm, H)
    y = or_ref[...].reshape(APM * bm, H)
    h = jnp.maximum(_dot_t(x, w1_ref[...]) + _dot_t(y, w2_ref[...])
                    + b_ref[...], 0.0)
    o_ref[...] = jnp.mean(h.reshape(APM, bm, H), axis=0)


def _readout(out_f, out_r, wo1, wo2, bias):
    bm = 128
    return pl.pallas_call(
        _readout_kernel,
        grid=(N_MOL // bm,),
        in_specs=[pl.BlockSpec((APM, bm, H), lambda i: (0, i, 0)),
                  pl.BlockSpec((APM, bm, H), lambda i: (0, i, 0)),
                  pl.BlockSpec((H, H), lambda i: (0, 0)),
                  pl.BlockSpec((H, H), lambda i: (0, 0)),
                  pl.BlockSpec((1, H), lambda i: (0, 0))],
        out_specs=pl.BlockSpec((bm, H), lambda i: (i, 0)),
        out_shape=jax.ShapeDtypeStruct((N_MOL, H), jnp.float32),
    )(out_f, out_r, wo1, wo2, bias)


def kernel(f_atoms, f_bonds, a2b, b2a, b2revb, a_scope, W_i_atom, W_i_bond,
           W_h_0, W_h_1, lr_W, W_o_W, W_o_b, gru_bias, W_ih_f, W_hh_f,
           b_ih_f, b_hh_f, W_ih_r, W_hh_r, b_ih_r, b_hh_r):
    del a_scope
    f32, i32 = jnp.float32, jnp.int32

    fb = jnp.concatenate(
        [f_bonds, jnp.zeros((PAD_B - N_BONDS, f_bonds.shape[1]), f32)],
        axis=0)
    a2b_r = jnp.concatenate(
        [a2b[1:], a2b[0:1], jnp.zeros((PAD_A - N_ATOMS, MAX_NB), a2b.dtype)],
        axis=0)
    a2bt = a2b_r.T.astype(i32)
    ar = jnp.arange(PAD_A, dtype=i32)
    r2a = jnp.where(ar < ORIG0, ar + 1, 0)
    ident = ar
    b2a_s = jnp.where(b2a == 0, ORIG0, b2a - 1).astype(i32)
    b2a_p = jnp.concatenate(
        [b2a_s, jnp.zeros((PAD_B - N_BONDS,), i32)], axis=0)
    b2revb_p = jnp.concatenate(
        [b2revb.astype(i32), jnp.zeros((PAD_B - N_BONDS,), i32)], axis=0)

    wo1 = W_o_W[:, :H]
    wo2 = W_o_W[:, H:]
    bias2 = gru_bias.reshape(1, H)
    wob2 = W_o_b.reshape(1, H)
    bih_f2 = b_ih_f.reshape(1, 3 * H)
    bhh_f2 = b_hh_f.reshape(1, 3 * H)
    bih_r2 = b_ih_r.reshape(1, 3 * H)
    bhh_r2 = b_hh_r.reshape(1, 3 * H)

    ia = _input_proj(f_atoms, W_i_atom, 512, n_out=PAD_A)
    ib = _input_proj(fb, W_i_bond, 2048)

    ma, mb = ia, ib
    for w_h, ridx in ((W_h_0, r2a), (W_h_1, ident)):
        ma = _sc_atom_agg(mb, a2bt, ma, ridx)
        pre = _sc_bond_msg(ma, mb, b2a_p, b2revb_p)
        mb = _bond_update(pre, ib, w_h)
    agg3, ia_roll = _sc_atom_agg_raw(mb, a2bt, ia, r2a)

    seq, h0 = _node_stage(agg3, ma, ia_roll, lr_W, bias2)
    out_f, out_r = _gru(seq, h0, W_ih_f, W_hh_f, bih_f2, bhh_f2,
                        W_ih_r, W_hh_r, bih_r2, bhh_r2)
    return _readout(out_f, out_r, wo1, wo2, wob2)


# speedup vs baseline: 1.0004x; 1.0004x over previous
import functools

import jax
import jax.numpy as jnp
from jax import lax
from jax.experimental import pallas as pl
from jax.experimental.pallas import tpu as pltpu
from jax.experimental.pallas import tpu_sc as plsc

H = 128
N_MOL = 1024
APM = 48
N_ATOMS = 1 + N_MOL * APM
N_BONDS = 1 + N_MOL * APM * 4
MAX_NB = 6
REAL_A = N_MOL * APM
ORIG0 = REAL_A

NC, NS = 2, 16
NW = NC * NS

CA = 128
PAD_A = 49280
NCHT_A = PAD_A // CA

CB = 128
PAD_B = 196736
NCHT_B = PAD_B // CB

_MESH = plsc.VectorSubcoreMesh(core_axis_name="c", subcore_axis_name="s")


def _wid():
    return lax.axis_index("s") * NC + lax.axis_index("c")


def _sc_atom_agg_body(mb, a2bt, ma, out, idx_v, rows_v, ma_v, sem):
    w = _wid()
    nch = (NCHT_A - w + NW - 1) // NW

    def chunk(ci, _):
        ao = (w + ci * NW) * CA
        pltpu.sync_copy(a2bt.at[:, pl.ds(ao, CA)], idx_v)
        pltpu.sync_copy(ma.at[pl.ds(ao, CA)], ma_v)
        cps = [pltpu.async_copy(mb.at[idx_v.at[j]], rows_v.at[j], sem)
               for j in range(MAX_NB)]
        for cp in cps:
            cp.wait()

        def row(r, _):
            for k in range(H // 16):
                sl = pl.ds(k * 16, 16)
                v = rows_v[0, r, sl]
                s_, m_ = v, v
                for j in range(1, MAX_NB):
                    vj = rows_v[j, r, sl]
                    s_ = s_ + vj
                    m_ = jnp.maximum(m_, vj)
                ma_v[r, sl] = ma_v[r, sl] + s_ * m_
            return 0

        lax.fori_loop(0, CA, row, 0)
        pltpu.sync_copy(ma_v, out.at[pl.ds(ao, CA)])
        return 0

    lax.fori_loop(0, nch, chunk, 0)


_sc_atom_agg = functools.partial(
    pl.kernel,
    out_type=jax.ShapeDtypeStruct((PAD_A, H), jnp.float32),
    mesh=_MESH,
    scratch_types=[
        pltpu.VMEM((MAX_NB, CA), jnp.int32),
        pltpu.VMEM((MAX_NB, CA, H), jnp.float32),
        pltpu.VMEM((CA, H), jnp.float32),
        pltpu.SemaphoreType.DMA,
    ],
)(_sc_atom_agg_body)


def _sc_atom_agg_raw_body(mb, a2bt, out, idx_v, rows_v, out_v, sem):
    w = _wid()
    nch = (NCHT_A - w + NW - 1) // NW

    def chunk(ci, _):
        ao = (w + ci * NW) * CA
        pltpu.sync_copy(a2bt.at[:, pl.ds(ao, CA)], idx_v)
        cps = [pltpu.async_copy(mb.at[idx_v.at[j]], rows_v.at[j], sem)
               for j in range(MAX_NB)]
        for cp in cps:
            cp.wait()

        def row(r, _):
            for k in range(H // 16):
                sl = pl.ds(k * 16, 16)
                v = rows_v[0, r, sl]
                s_, m_ = v, v
                for j in range(1, MAX_NB):
                    vj = rows_v[j, r, sl]
                    s_ = s_ + vj
                    m_ = jnp.maximum(m_, vj)
                out_v[r, sl] = s_ * m_
            return 0

        lax.fori_loop(0, CA, row, 0)
        pltpu.sync_copy(out_v, out.at[pl.ds(ao, CA)])
        return 0

    lax.fori_loop(0, nch, chunk, 0)


_sc_atom_agg_raw = functools.partial(
    pl.kernel,
    out_type=jax.ShapeDtypeStruct((PAD_A, H), jnp.float32),
    mesh=_MESH,
    scratch_types=[
        pltpu.VMEM((MAX_NB, CA), jnp.int32),
        pltpu.VMEM((MAX_NB, CA, H), jnp.float32),
        pltpu.VMEM((CA, H), jnp.float32),
        pltpu.SemaphoreType.DMA,
    ],
)(_sc_atom_agg_raw_body)


def _sc_bond_msg_body(ma, mb, b2a, b2revb, out, ia_v, ir_v, ra_v, rb_v,
                      out_v, sem):
    w = _wid()
    nch = (NCHT_B - w + NW - 1) // NW

    def chunk(ci, _):
        bo = (w + ci * NW) * CB
        pltpu.sync_copy(b2a.at[pl.ds(bo, CB)], ia_v)
        pltpu.sync_copy(b2revb.at[pl.ds(bo, CB)], ir_v)
        c1 = pltpu.async_copy(ma.at[ia_v], ra_v, sem)
        c2 = pltpu.async_copy(mb.at[ir_v], rb_v, sem)
        c1.wait()
        c2.wait()

        def row(r, _):
            for k in range(H // 16):
                sl = pl.ds(k * 16, 16)
                out_v[r, sl] = ra_v[r, sl] - rb_v[r, sl]
            return 0

        lax.fori_loop(0, CB, row, 0)
        pltpu.sync_copy(out_v, out.at[pl.ds(bo, CB)])
        return 0

    lax.fori_loop(0, nch, chunk, 0)


_sc_bond_msg = functools.partial(
    pl.kernel,
    out_type=jax.ShapeDtypeStruct((PAD_B, H), jnp.float32),
    mesh=_MESH,
    scratch_types=[
        pltpu.VMEM((CB,), jnp.int32),
        pltpu.VMEM((CB,), jnp.int32),
        pltpu.VMEM((CB, H), jnp.float32),
        pltpu.VMEM((CB, H), jnp.float32),
        pltpu.VMEM((CB, H), jnp.float32),
        pltpu.SemaphoreType.DMA,
    ],
)(_sc_bond_msg_body)


def _dot_t(x, w):
    return lax.dot_general(x, w, (((1,), (1,)), ((), ())),
                           precision=None,
                           preferred_element_type=jnp.float32)


def _relu_mm_kernel(x_ref, w_ref, o_ref):
    o_ref[...] = jnp.maximum(_dot_t(x_ref[...], w_ref[...]), 0.0)


def _input_proj(x, w, br, n_out=None):
    n, k = x.shape
    if n_out is None:
        n_out = n
    return pl.pallas_call(
        _relu_mm_kernel,
        grid=(pl.cdiv(n_out, br),),
        in_specs=[pl.BlockSpec((br, k), lambda i: (i, 0)),
                  pl.BlockSpec((H, k), lambda i: (0, 0))],
        out_specs=pl.BlockSpec((br, H), lambda i: (i, 0)),
        out_shape=jax.ShapeDtypeStruct((n_out, H), jnp.float32),
    )(x, w)


def _bond_update_kernel(pre_ref, ib_ref, w_ref, o_ref):
    o_ref[...] = jnp.maximum(
        ib_ref[...] + _dot_t(pre_ref[...], w_ref[...]), 0.0)


def _bond_update(pre, ib, w_h):
    br = 2048
    return pl.pallas_call(
        _bond_update_kernel,
        grid=(pl.cdiv(PAD_B, br),),
        in_specs=[pl.BlockSpec((br, H), lambda i: (i, 0)),
                  pl.BlockSpec((br, H), lambda i: (i, 0)),
                  pl.BlockSpec((H, H), lambda i: (0, 0))],
        out_specs=pl.BlockSpec((br, H), lambda i: (i, 0)),
        out_shape=jax.ShapeDtypeStruct((PAD_B, H), jnp.float32),
    )(pre, ib, w_h)


def _node_kernel(agg_ref, ma_ref, ia_ref, w_ref, bias_ref, msg_ref, h0_ref):
    w = w_ref[...]
    node = (_dot_t(agg_ref[...], w[:, :H])
            + _dot_t(ma_ref[...], w[:, H:2 * H])
            + _dot_t(ia_ref[...], w[:, 2 * H:]))
    msg = jnp.maximum(node + bias_ref[...], 0.0)
    msg_ref[...] = msg.reshape(-1, APM, H).swapaxes(0, 1)
    h0_ref[...] = jnp.max(node.reshape(-1, APM, H), axis=1)


def _node_stage(agg, ma, ia, lr_w, bias):
    br = 768
    mols = br // APM
    return pl.pallas_call(
        _node_kernel,
        grid=(REAL_A // br,),
        in_specs=[pl.BlockSpec((br, H), lambda i: (i, 0)),
                  pl.BlockSpec((br, H), lambda i: (i, 0)),
                  pl.BlockSpec((br, H), lambda i: (i, 0)),
                  pl.BlockSpec((H, 3 * H), lambda i: (0, 0)),
                  pl.BlockSpec((1, H), lambda i: (0, 0))],
        out_specs=[pl.BlockSpec((APM, mols, H), lambda i: (0, i, 0)),
                   pl.BlockSpec((mols, H), lambda i: (i, 0))],
        out_shape=[jax.ShapeDtypeStruct((APM, N_MOL, H), jnp.float32),
                   jax.ShapeDtypeStruct((N_MOL, H), jnp.float32)],
    )(agg, ma, ia, lr_w, bias)


def _gru_step(x, h, wih, whh, bih, bhh):
    gi = _dot_t(x, wih) + bih
    gh = _dot_t(h, whh) + bhh
    r = jax.nn.sigmoid(gi[:, :H] + gh[:, :H])
    z = jax.nn.sigmoid(gi[:, H:2 * H] + gh[:, H:2 * H])
    n = jnp.tanh(gi[:, 2 * H:] + r * gh[:, 2 * H:])
    return (1.0 - z) * n + z * h


def _gru_kernel(xf_ref, xr_ref, h0_ref, wih_f, whh_f, bih_f, bhh_f,
                wih_r, whh_r, bih_r, bhh_r, of_ref, or_ref, hf, hr):
    t = pl.program_id(0)

    @pl.when(t == 0)
    def _():
        hf[...] = h0_ref[...]
        hr[...] = h0_ref[...]

    xf = xf_ref[...][0]
    hn_f = _gru_step(xf, hf[...], wih_f[...], whh_f[...], bih_f[...],
                     bhh_f[...])
    hf[...] = hn_f
    of_ref[...] = hn_f[None]

    xr = xr_ref[...][0]
    hn_r = _gru_step(xr, hr[...], wih_r[...], whh_r[...], bih_r[...],
                     bhh_r[...])
    hr[...] = hn_r
    or_ref[...] = hn_r[None]


def _gru(seq, h0, wih_f, whh_f, bih_f, bhh_f, wih_r, whh_r, bih_r, bhh_r):
    wspec = pl.BlockSpec((3 * H, H), lambda t: (0, 0))
    bspec = pl.BlockSpec((1, 3 * H), lambda t: (0, 0))
    return pl.pallas_call(
        _gru_kernel,
        grid=(APM,),
        in_specs=[pl.BlockSpec((1, N_MOL, H), lambda t: (t, 0, 0)),
                  pl.BlockSpec((1, N_MOL, H), lambda t: (APM - 1 - t, 0, 0)),
                  pl.BlockSpec((N_MOL, H), lambda t: (0, 0)),
                  wspec, wspec, bspec, bspec, wspec, wspec, bspec, bspec],
        out_specs=[pl.BlockSpec((1, N_MOL, H), lambda t: (t, 0, 0)),
                   pl.BlockSpec((1, N_MOL, H),
                                lambda t: (APM - 1 - t, 0, 0))],
        out_shape=[jax.ShapeDtypeStruct((APM, N_MOL, H), jnp.float32),
                   jax.ShapeDtypeStruct((APM, N_MOL, H), jnp.float32)],
        scratch_shapes=[pltpu.VMEM((N_MOL, H), jnp.float32),
                        pltpu.VMEM((N_MOL, H), jnp.float32)],
        compiler_params=pltpu.CompilerParams(
            dimension_semantics=("arbitrary",)),
    )(seq, seq, h0, wih_f, whh_f, bih_f, bhh_f, wih_r, whh_r, bih_r, bhh_r)


def _readout_kernel(of_ref, or_ref, w1_ref, w2_ref, b_ref, o_ref):
    bm = of_ref.shape[1]
    x = of_ref[...].reshape(APM * bm, H)
    y = or_ref[...].reshape(APM * bm, H)
    h = jnp.maximum(_dot_t(x, w1_ref[...]) + _dot_t(y, w2_ref[...])
                    + b_ref[...], 0.0)
    o_ref[...] = jnp.mean(h.reshape(APM, bm, H), axis=0)


def _readout(out_f, out_r, wo1, wo2, bias):
    bm = 128
    return pl.pallas_call(
        _readout_kernel,
        grid=(N_MOL // bm,),
        in_specs=[pl.BlockSpec((APM, bm, H), lambda i: (0, i, 0)),
                  pl.BlockSpec((APM, bm, H), lambda i: (0, i, 0)),
                  pl.BlockSpec((H, H), lambda i: (0, 0)),
                  pl.BlockSpec((H, H), lambda i: (0, 0)),
                  pl.BlockSpec((1, H), lambda i: (0, 0))],
        out_specs=pl.BlockSpec((bm, H), lambda i: (i, 0)),
        out_shape=jax.ShapeDtypeStruct((N_MOL, H), jnp.float32),
    )(out_f, out_r, wo1, wo2, bias)


def kernel(f_atoms, f_bonds, a2b, b2a, b2revb, a_scope, W_i_atom, W_i_bond,
           W_h_0, W_h_1, lr_W, W_o_W, W_o_b, gru_bias, W_ih_f, W_hh_f,
           b_ih_f, b_hh_f, W_ih_r, W_hh_r, b_ih_r, b_hh_r):
    del a_scope
    f32, i32 = jnp.float32, jnp.int32

    fa = jnp.concatenate(
        [f_atoms[1:], f_atoms[0:1],
         jnp.zeros((PAD_A - N_ATOMS, f_atoms.shape[1]), f32)], axis=0)
    fb = jnp.concatenate(
        [f_bonds, jnp.zeros((PAD_B - N_BONDS, f_bonds.shape[1]), f32)],
        axis=0)
    a2b_r = jnp.concatenate(
        [a2b[1:], a2b[0:1], jnp.zeros((PAD_A - N_ATOMS, MAX_NB), a2b.dtype)],
        axis=0)
    a2bt = a2b_r.T.astype(i32)
    b2a_s = jnp.where(b2a == 0, ORIG0, b2a - 1).astype(i32)
    b2a_p = jnp.concatenate(
        [b2a_s, jnp.zeros((PAD_B - N_BONDS,), i32)], axis=0)
    b2revb_p = jnp.concatenate(
        [b2revb.astype(i32), jnp.zeros((PAD_B - N_BONDS,), i32)], axis=0)

    wo1 = W_o_W[:, :H]
    wo2 = W_o_W[:, H:]
    bias2 = gru_bias.reshape(1, H)
    wob2 = W_o_b.reshape(1, H)
    bih_f2 = b_ih_f.reshape(1, 3 * H)
    bhh_f2 = b_hh_f.reshape(1, 3 * H)
    bih_r2 = b_ih_r.reshape(1, 3 * H)
    bhh_r2 = b_hh_r.reshape(1, 3 * H)

    ia = _input_proj(fa, W_i_atom, 512)
    ib = _input_proj(fb, W_i_bond, 2048)

    ma, mb = ia, ib
    for w_h in (W_h_0, W_h_1):
        ma = _sc_atom_agg(mb, a2bt, ma)
        pre = _sc_bond_msg(ma, mb, b2a_p, b2revb_p)
        mb = _bond_update(pre, ib, w_h)
    agg3 = _sc_atom_agg_raw(mb, a2bt)

    seq, h0 = _node_stage(agg3, ma, ia, lr_W, bias2)
    out_f, out_r = _gru(seq, h0, W_ih_f, W_hh_f, bih_f2, bhh_f2,
                        W_ih_r, W_hh_r, bih_r2, bhh_r2)
    return _readout(out_f, out_r, wo1, wo2, wob2)
